# baseline (speedup 1.0000x reference)
import functools
import math

import jax
import jax.numpy as jnp
from jax import lax
from jax.experimental import pallas as pl
from jax.experimental.pallas import tpu as pltpu
from jax.experimental.pallas import tpu_sc as plsc

VOCAB = 100000
EMBED = 128
B = 1024
L = 200
LANES = 16
CHUNKS_PER_ROW = EMBED // LANES
GCHUNK = 40
PE_PERIOD = L // GCHUNK
NBUF = 6
LOOKAHEAD = 2


def _positional_embedding(seq_len, d_model):
    position = jnp.arange(0, seq_len, dtype=jnp.float32)[:, None]
    div_term = jnp.exp(
        jnp.arange(0, d_model, 2, dtype=jnp.float32) * -(math.log(10000.0) / d_model)
    )
    pe = jnp.zeros((seq_len, d_model), dtype=jnp.float32)
    pe = pe.at[:, 0::2].set(jnp.sin(position * div_term))
    pe = pe.at[:, 1::2].set(jnp.cos(position * div_term))
    return pe


def _make_sc_kernel(n_workers):
    n_chunks = (B * L) // GCHUNK
    chunks_per_w = n_chunks // n_workers
    mesh = plsc.VectorSubcoreMesh(core_axis_name="c", subcore_axis_name="s")
    num_cores = mesh.num_cores

    @functools.partial(
        pl.kernel,
        mesh=mesh,
        out_type=jax.ShapeDtypeStruct((B * L, EMBED), jnp.float32),
        scratch_types=(
            [pltpu.VMEM((chunks_per_w, GCHUNK), jnp.int32)]
            + [pltpu.VMEM((L, EMBED), jnp.float32)]
            + [pltpu.VMEM((GCHUNK, EMBED), jnp.float32)] * NBUF
            + [pltpu.SemaphoreType.DMA] * (2 * NBUF)
        ),
    )
    def k(seq_hbm, table_hbm, pe_hbm, out_hbm, idx_v, pe_v, *bufs_sems):
        rows = bufs_sems[:NBUF]
        gsem = bufs_sems[NBUF : 2 * NBUF]
        osem = bufs_sems[2 * NBUF :]
        wid = lax.axis_index("s") * num_cores + lax.axis_index("c")
        chunk_base = wid * chunks_per_w
        pltpu.sync_copy(pe_hbm, pe_v)
        pltpu.sync_copy(seq_hbm.at[pl.ds(chunk_base, chunks_per_w)], idx_v)

        def gather_start(c, b):
            pltpu.make_async_copy(
                table_hbm.at[idx_v.at[c]], rows[b], gsem[b]
            ).start()

        def gather_wait(c, b):
            pltpu.make_async_copy(
                table_hbm.at[idx_v.at[c]], rows[b], gsem[b]
            ).wait()

        def add_pe(c, b):
            pe_off = lax.rem(c, PE_PERIOD) * GCHUNK

            def add_row(r, c2):
                for j in range(CHUNKS_PER_ROW):
                    sl = pl.ds(j * LANES, LANES)
                    rows[b][r, sl] = rows[b][r, sl] + pe_v[pe_off + r, sl]
                return c2

            lax.fori_loop(0, GCHUNK, add_row, 0, unroll=8)

        def out_start(c, b):
            pltpu.make_async_copy(
                rows[b], out_hbm.at[pl.ds((chunk_base + c) * GCHUNK, GCHUNK)], osem[b]
            ).start()

        def out_wait(b):
            pltpu.make_async_copy(
                rows[b], out_hbm.at[pl.ds(0, GCHUNK)], osem[b]
            ).wait()

        def slot(c, pb, bslot, wait_out, issue):
            if issue:
                if wait_out:
                    out_wait(bslot)
                gather_start(c + LOOKAHEAD, bslot)
            gather_wait(c, pb)
            add_pe(c, pb)
            out_start(c, pb)

        for b in range(LOOKAHEAD):
            gather_start(b, b)

        c0 = NBUF - LOOKAHEAD
        for c in range(c0):
            slot(c, c % NBUF, (c + LOOKAHEAD) % NBUF, False, True)

        n_main = (chunks_per_w - LOOKAHEAD - c0) // NBUF

        def group(g0, carry):
            for b in range(NBUF):
                c = c0 + g0 * NBUF + b
                slot(c, (c0 + b) % NBUF, (c0 + b + LOOKAHEAD) % NBUF, True, True)
            return carry

        lax.fori_loop(0, n_main, group, 0)

        for c in range(c0 + n_main * NBUF, chunks_per_w):
            slot(c, c % NBUF, (c + LOOKAHEAD) % NBUF,
                 c + LOOKAHEAD >= NBUF, c + LOOKAHEAD < chunks_per_w)

        for b in range(NBUF):
            out_wait(b)

    return k


def kernel(sequence, token_table):
    n_chunks = (B * L) // GCHUNK
    seq = sequence.astype(jnp.int32).reshape(n_chunks, GCHUNK)
    pe = _positional_embedding(L, EMBED)
    info = plsc.get_sparse_core_info()
    n_workers = info.num_cores * info.num_subcores
    out = _make_sc_kernel(n_workers)(seq, token_table, pe)
    return out.reshape(B, L, EMBED)


# speedup vs baseline: 1.9908x; 1.9908x over previous
import functools
import math

import jax
import jax.numpy as jnp
from jax import lax
from jax.experimental import pallas as pl
from jax.experimental.pallas import tpu as pltpu
from jax.experimental.pallas import tpu_sc as plsc

VOCAB = 100000
EMBED = 128
B = 1024
L = 200
LANES = 16
CHUNKS_PER_ROW = EMBED // LANES
GCHUNK = 40
PE_PERIOD = L // GCHUNK
NBUF = 6
LOOKAHEAD = 2


def _positional_embedding(seq_len, d_model):
    position = jnp.arange(0, seq_len, dtype=jnp.float32)[:, None]
    div_term = jnp.exp(
        jnp.arange(0, d_model, 2, dtype=jnp.float32) * -(math.log(10000.0) / d_model)
    )
    pe = jnp.zeros((seq_len, d_model), dtype=jnp.float32)
    pe = pe.at[:, 0::2].set(jnp.sin(position * div_term))
    pe = pe.at[:, 1::2].set(jnp.cos(position * div_term))
    return pe


def _make_sc_kernel(n_workers):
    n_chunks = (B * L) // GCHUNK
    chunks_per_w = n_chunks // n_workers
    mesh = plsc.VectorSubcoreMesh(core_axis_name="c", subcore_axis_name="s")
    num_cores = mesh.num_cores

    @functools.partial(
        pl.kernel,
        mesh=mesh,
        out_type=jax.ShapeDtypeStruct((B * L, EMBED), jnp.float32),
        scratch_types=(
            [pltpu.VMEM((chunks_per_w, GCHUNK), jnp.int32)]
            + [pltpu.VMEM((L, EMBED), jnp.float32)]
            + [pltpu.VMEM((GCHUNK, EMBED), jnp.float32)] * NBUF
            + [pltpu.SemaphoreType.DMA] * (2 * NBUF)
        ),
    )
    def k(seq_hbm, table_hbm, pe_hbm, out_hbm, idx_v, pe_v, *bufs_sems):
        rows = bufs_sems[:NBUF]
        gsem = bufs_sems[NBUF : 2 * NBUF]
        osem = bufs_sems[2 * NBUF :]
        wid = lax.axis_index("s") * num_cores + lax.axis_index("c")
        chunk_base = wid * chunks_per_w
        pltpu.sync_copy(pe_hbm, pe_v)
        pltpu.sync_copy(seq_hbm.at[pl.ds(chunk_base, chunks_per_w)], idx_v)

        def gather_start(c, b):
            pltpu.make_async_copy(
                table_hbm.at[idx_v.at[c]], rows[b], gsem[b]
            ).start()

        def gather_wait(c, b):
            pltpu.make_async_copy(
                table_hbm.at[idx_v.at[c]], rows[b], gsem[b]
            ).wait()

        def add_pe(c, b):
            pe_off = lax.rem(c, PE_PERIOD) * GCHUNK

            @plsc.parallel_loop(0, GCHUNK, unroll=8)
            def add_row(r):
                for j in range(CHUNKS_PER_ROW):
                    sl = pl.ds(j * LANES, LANES)
                    plsc.addupdate(rows[b].at[r, sl], pe_v[pe_off + r, sl])

        def out_start(c, b):
            pltpu.make_async_copy(
                rows[b], out_hbm.at[pl.ds((chunk_base + c) * GCHUNK, GCHUNK)], osem[b]
            ).start()

        def out_wait(b):
            pltpu.make_async_copy(
                rows[b], out_hbm.at[pl.ds(0, GCHUNK)], osem[b]
            ).wait()

        def slot(c, pb, bslot, wait_out, issue):
            if issue:
                if wait_out:
                    out_wait(bslot)
                gather_start(c + LOOKAHEAD, bslot)
            gather_wait(c, pb)
            add_pe(c, pb)
            out_start(c, pb)

        for b in range(LOOKAHEAD):
            gather_start(b, b)

        c0 = NBUF - LOOKAHEAD
        for c in range(c0):
            slot(c, c % NBUF, (c + LOOKAHEAD) % NBUF, False, True)

        n_main = (chunks_per_w - LOOKAHEAD - c0) // NBUF

        def group(g0, carry):
            for b in range(NBUF):
                c = c0 + g0 * NBUF + b
                slot(c, (c0 + b) % NBUF, (c0 + b + LOOKAHEAD) % NBUF, True, True)
            return carry

        lax.fori_loop(0, n_main, group, 0)

        for c in range(c0 + n_main * NBUF, chunks_per_w):
            slot(c, c % NBUF, (c + LOOKAHEAD) % NBUF,
                 c + LOOKAHEAD >= NBUF, c + LOOKAHEAD < chunks_per_w)

        for b in range(NBUF):
            out_wait(b)

    return k


def kernel(sequence, token_table):
    n_chunks = (B * L) // GCHUNK
    seq = sequence.astype(jnp.int32).reshape(n_chunks, GCHUNK)
    pe = _positional_embedding(L, EMBED)
    info = plsc.get_sparse_core_info()
    n_workers = info.num_cores * info.num_subcores
    out = _make_sc_kernel(n_workers)(seq, token_table, pe)
    return out.reshape(B, L, EMBED)


# speedup vs baseline: 2.3025x; 1.1566x over previous
import functools
import math

import jax
import jax.numpy as jnp
from jax import lax
from jax.experimental import pallas as pl
from jax.experimental.pallas import tpu as pltpu
from jax.experimental.pallas import tpu_sc as plsc

VOCAB = 100000
EMBED = 128
B = 1024
L = 200
LANES = 16
CHUNKS_PER_ROW = EMBED // LANES
GCHUNK = 40
PE_PERIOD = L // GCHUNK
NBUF = 6
LOOKAHEAD = 2


def _positional_embedding(seq_len, d_model):
    position = jnp.arange(0, seq_len, dtype=jnp.float32)[:, None]
    div_term = jnp.exp(
        jnp.arange(0, d_model, 2, dtype=jnp.float32) * -(math.log(10000.0) / d_model)
    )
    pe = jnp.zeros((seq_len, d_model), dtype=jnp.float32)
    pe = pe.at[:, 0::2].set(jnp.sin(position * div_term))
    pe = pe.at[:, 1::2].set(jnp.cos(position * div_term))
    return pe


def _make_sc_kernel(n_workers):
    n_chunks = (B * L) // GCHUNK
    chunks_per_w = n_chunks // n_workers
    mesh = plsc.VectorSubcoreMesh(core_axis_name="c", subcore_axis_name="s")
    num_cores = mesh.num_cores

    @functools.partial(
        pl.kernel,
        mesh=mesh,
        out_type=jax.ShapeDtypeStruct((B * L, EMBED), jnp.float32),
        scratch_types=(
            [pltpu.VMEM((chunks_per_w, GCHUNK), jnp.int32)]
            + [pltpu.VMEM((L, EMBED), jnp.float32)]
            + [pltpu.VMEM((GCHUNK, EMBED), jnp.float32)] * NBUF
            + [pltpu.SemaphoreType.DMA] * (2 * NBUF)
        ),
    )
    def k(seq_hbm, table_hbm, pe_hbm, out_hbm, idx_v, pe_v, *bufs_sems):
        rows = bufs_sems[:NBUF]
        gsem = bufs_sems[NBUF : 2 * NBUF]
        osem = bufs_sems[2 * NBUF :]
        wid = lax.axis_index("s") * num_cores + lax.axis_index("c")
        chunk_base = wid * chunks_per_w
        pltpu.sync_copy(pe_hbm, pe_v)
        pltpu.sync_copy(seq_hbm.at[pl.ds(chunk_base, chunks_per_w)], idx_v)

        def gather_start(c, b):
            pltpu.make_async_copy(
                table_hbm.at[idx_v.at[c]], rows[b], gsem[b]
            ).start()

        def gather_wait(c, b):
            pltpu.make_async_copy(
                table_hbm.at[idx_v.at[c]], rows[b], gsem[b]
            ).wait()

        def add_pe(c, b):
            pe_off = lax.rem(c, PE_PERIOD) * GCHUNK

            @plsc.parallel_loop(0, GCHUNK, unroll=4)
            def add_row(r):
                for j in range(CHUNKS_PER_ROW):
                    sl = pl.ds(j * LANES, LANES)
                    plsc.addupdate(rows[b].at[r, sl], pe_v[pe_off + r, sl])

        def out_start(c, b):
            pltpu.make_async_copy(
                rows[b], out_hbm.at[pl.ds((chunk_base + c) * GCHUNK, GCHUNK)], osem[b]
            ).start()

        def out_wait(b):
            pltpu.make_async_copy(
                rows[b], out_hbm.at[pl.ds(0, GCHUNK)], osem[b]
            ).wait()

        def slot(c, pb, bslot, wait_out, issue):
            if issue:
                if wait_out:
                    out_wait(bslot)
                gather_start(c + LOOKAHEAD, bslot)
            gather_wait(c, pb)
            add_pe(c, pb)
            out_start(c, pb)

        for b in range(LOOKAHEAD):
            gather_start(b, b)

        c0 = NBUF - LOOKAHEAD
        for c in range(c0):
            slot(c, c % NBUF, (c + LOOKAHEAD) % NBUF, False, True)

        n_main = (chunks_per_w - LOOKAHEAD - c0) // NBUF

        def group(g0, carry):
            for b in range(NBUF):
                c = c0 + g0 * NBUF + b
                slot(c, (c0 + b) % NBUF, (c0 + b + LOOKAHEAD) % NBUF, True, True)
            return carry

        lax.fori_loop(0, n_main, group, 0)

        for c in range(c0 + n_main * NBUF, chunks_per_w):
            slot(c, c % NBUF, (c + LOOKAHEAD) % NBUF,
                 c + LOOKAHEAD >= NBUF, c + LOOKAHEAD < chunks_per_w)

        for b in range(NBUF):
            out_wait(b)

    return k


def kernel(sequence, token_table):
    n_chunks = (B * L) // GCHUNK
    seq = sequence.astype(jnp.int32).reshape(n_chunks, GCHUNK)
    pe = _positional_embedding(L, EMBED)
    info = plsc.get_sparse_core_info()
    n_workers = info.num_cores * info.num_subcores
    out = _make_sc_kernel(n_workers)(seq, token_table, pe)
    return out.reshape(B, L, EMBED)


# speedup vs baseline: 2.3163x; 1.0060x over previous
import functools
import math

import jax
import jax.numpy as jnp
from jax import lax
from jax.experimental import pallas as pl
from jax.experimental.pallas import tpu as pltpu
from jax.experimental.pallas import tpu_sc as plsc

VOCAB = 100000
EMBED = 128
B = 1024
L = 200
LANES = 16
CHUNKS_PER_ROW = EMBED // LANES
GCHUNK = 40
PE_PERIOD = L // GCHUNK
NBUF = 6
LOOKAHEAD = 2


def _positional_embedding(seq_len, d_model):
    position = jnp.arange(0, seq_len, dtype=jnp.float32)[:, None]
    div_term = jnp.exp(
        jnp.arange(0, d_model, 2, dtype=jnp.float32) * -(math.log(10000.0) / d_model)
    )
    pe = jnp.zeros((seq_len, d_model), dtype=jnp.float32)
    pe = pe.at[:, 0::2].set(jnp.sin(position * div_term))
    pe = pe.at[:, 1::2].set(jnp.cos(position * div_term))
    return pe


def _make_sc_kernel(n_workers):
    n_chunks = (B * L) // GCHUNK
    chunks_per_w = n_chunks // n_workers
    mesh = plsc.VectorSubcoreMesh(core_axis_name="c", subcore_axis_name="s")
    num_cores = mesh.num_cores

    @functools.partial(
        pl.kernel,
        mesh=mesh,
        out_type=jax.ShapeDtypeStruct((B * L, EMBED), jnp.float32),
        scratch_types=(
            [pltpu.VMEM((chunks_per_w, GCHUNK), jnp.int32)]
            + [pltpu.VMEM((L, EMBED), jnp.float32)]
            + [pltpu.VMEM((GCHUNK, EMBED), jnp.float32)] * NBUF
            + [pltpu.SemaphoreType.DMA] * (2 * NBUF)
        ),
    )
    def k(seq_hbm, table_hbm, pe_hbm, out_hbm, idx_v, pe_v, *bufs_sems):
        rows = bufs_sems[:NBUF]
        gsem = bufs_sems[NBUF : 2 * NBUF]
        osem = bufs_sems[2 * NBUF :]
        wid = lax.axis_index("s") * num_cores + lax.axis_index("c")
        chunk_base = wid * chunks_per_w
        pltpu.sync_copy(pe_hbm, pe_v)
        pltpu.sync_copy(seq_hbm.at[pl.ds(chunk_base, chunks_per_w)], idx_v)

        def gather_start(c, b):
            pltpu.make_async_copy(
                table_hbm.at[idx_v.at[c]], rows[b], gsem[b]
            ).start()

        def gather_wait(c, b):
            pltpu.make_async_copy(
                table_hbm.at[idx_v.at[c]], rows[b], gsem[b]
            ).wait()

        def add_pe(c, b):
            pe_off = lax.rem(c, PE_PERIOD) * GCHUNK

            @plsc.parallel_loop(0, GCHUNK, unroll=2)
            def add_row(r):
                for j in range(CHUNKS_PER_ROW):
                    sl = pl.ds(j * LANES, LANES)
                    plsc.addupdate(rows[b].at[r, sl], pe_v[pe_off + r, sl])

        def out_start(c, b):
            pltpu.make_async_copy(
                rows[b], out_hbm.at[pl.ds((chunk_base + c) * GCHUNK, GCHUNK)], osem[b]
            ).start()

        def out_wait(b):
            pltpu.make_async_copy(
                rows[b], out_hbm.at[pl.ds(0, GCHUNK)], osem[b]
            ).wait()

        def slot(c, pb, bslot, wait_out, issue):
            if issue:
                if wait_out:
                    out_wait(bslot)
                gather_start(c + LOOKAHEAD, bslot)
            gather_wait(c, pb)
            add_pe(c, pb)
            out_start(c, pb)

        for b in range(LOOKAHEAD):
            gather_start(b, b)

        c0 = NBUF - LOOKAHEAD
        for c in range(c0):
            slot(c, c % NBUF, (c + LOOKAHEAD) % NBUF, False, True)

        n_main = (chunks_per_w - LOOKAHEAD - c0) // NBUF

        def group(g0, carry):
            for b in range(NBUF):
                c = c0 + g0 * NBUF + b
                slot(c, (c0 + b) % NBUF, (c0 + b + LOOKAHEAD) % NBUF, True, True)
            return carry

        lax.fori_loop(0, n_main, group, 0)

        for c in range(c0 + n_main * NBUF, chunks_per_w):
            slot(c, c % NBUF, (c + LOOKAHEAD) % NBUF,
                 c + LOOKAHEAD >= NBUF, c + LOOKAHEAD < chunks_per_w)

        for b in range(NBUF):
            out_wait(b)

    return k


def kernel(sequence, token_table):
    n_chunks = (B * L) // GCHUNK
    seq = sequence.astype(jnp.int32).reshape(n_chunks, GCHUNK)
    pe = _positional_embedding(L, EMBED)
    info = plsc.get_sparse_core_info()
    n_workers = info.num_cores * info.num_subcores
    out = _make_sc_kernel(n_workers)(seq, token_table, pe)
    return out.reshape(B, L, EMBED)


# speedup vs baseline: 2.3199x; 1.0015x over previous
import functools
import math

import jax
import jax.numpy as jnp
from jax import lax
from jax.experimental import pallas as pl
from jax.experimental.pallas import tpu as pltpu
from jax.experimental.pallas import tpu_sc as plsc

VOCAB = 100000
EMBED = 128
B = 1024
L = 200
LANES = 16
CHUNKS_PER_ROW = EMBED // LANES
GCHUNK = 40
PE_PERIOD = L // GCHUNK
NBUF = 6
LOOKAHEAD = 2


def _positional_embedding(seq_len, d_model):
    position = jnp.arange(0, seq_len, dtype=jnp.float32)[:, None]
    div_term = jnp.exp(
        jnp.arange(0, d_model, 2, dtype=jnp.float32) * -(math.log(10000.0) / d_model)
    )
    pe = jnp.zeros((seq_len, d_model), dtype=jnp.float32)
    pe = pe.at[:, 0::2].set(jnp.sin(position * div_term))
    pe = pe.at[:, 1::2].set(jnp.cos(position * div_term))
    return pe


def _make_sc_kernel(n_workers):
    n_chunks = (B * L) // GCHUNK
    chunks_per_w = n_chunks // n_workers
    mesh = plsc.VectorSubcoreMesh(core_axis_name="c", subcore_axis_name="s")
    num_cores = mesh.num_cores

    @functools.partial(
        pl.kernel,
        mesh=mesh,
        out_type=jax.ShapeDtypeStruct((B * L, EMBED), jnp.float32),
        scratch_types=(
            [pltpu.VMEM((chunks_per_w, GCHUNK), jnp.int32)]
            + [pltpu.VMEM((L, EMBED), jnp.float32)]
            + [pltpu.VMEM((GCHUNK, EMBED), jnp.float32)] * NBUF
            + [pltpu.SemaphoreType.DMA] * (2 * NBUF)
        ),
    )
    def k(seq_hbm, table_hbm, pe_hbm, out_hbm, idx_v, pe_v, *bufs_sems):
        rows = bufs_sems[:NBUF]
        gsem = bufs_sems[NBUF : 2 * NBUF]
        osem = bufs_sems[2 * NBUF :]
        wid = lax.axis_index("s") * num_cores + lax.axis_index("c")
        chunk_base = wid * chunks_per_w
        pltpu.sync_copy(pe_hbm, pe_v)
        pltpu.sync_copy(seq_hbm.at[pl.ds(chunk_base, chunks_per_w)], idx_v)

        def gather_start(c, b):
            pltpu.make_async_copy(
                table_hbm.at[idx_v.at[c]], rows[b], gsem[b]
            ).start()

        def gather_wait(c, b):
            pltpu.make_async_copy(
                table_hbm.at[idx_v.at[c]], rows[b], gsem[b]
            ).wait()

        def add_pe(c, b):
            pe_off = lax.rem(c, PE_PERIOD) * GCHUNK

            @plsc.parallel_loop(0, GCHUNK, unroll=1)
            def add_row(r):
                for j in range(CHUNKS_PER_ROW):
                    sl = pl.ds(j * LANES, LANES)
                    plsc.addupdate(rows[b].at[r, sl], pe_v[pe_off + r, sl])

        def out_start(c, b):
            pltpu.make_async_copy(
                rows[b], out_hbm.at[pl.ds((chunk_base + c) * GCHUNK, GCHUNK)], osem[b]
            ).start()

        def out_wait(b):
            pltpu.make_async_copy(
                rows[b], out_hbm.at[pl.ds(0, GCHUNK)], osem[b]
            ).wait()

        def slot(c, pb, bslot, wait_out, issue):
            if issue:
                if wait_out:
                    out_wait(bslot)
                gather_start(c + LOOKAHEAD, bslot)
            gather_wait(c, pb)
            add_pe(c, pb)
            out_start(c, pb)

        for b in range(LOOKAHEAD):
            gather_start(b, b)

        c0 = NBUF - LOOKAHEAD
        for c in range(c0):
            slot(c, c % NBUF, (c + LOOKAHEAD) % NBUF, False, True)

        n_main = (chunks_per_w - LOOKAHEAD - c0) // NBUF

        def group(g0, carry):
            for b in range(NBUF):
                c = c0 + g0 * NBUF + b
                slot(c, (c0 + b) % NBUF, (c0 + b + LOOKAHEAD) % NBUF, True, True)
            return carry

        lax.fori_loop(0, n_main, group, 0)

        for c in range(c0 + n_main * NBUF, chunks_per_w):
            slot(c, c % NBUF, (c + LOOKAHEAD) % NBUF,
                 c + LOOKAHEAD >= NBUF, c + LOOKAHEAD < chunks_per_w)

        for b in range(NBUF):
            out_wait(b)

    return k


def kernel(sequence, token_table):
    n_chunks = (B * L) // GCHUNK
    seq = sequence.astype(jnp.int32).reshape(n_chunks, GCHUNK)
    pe = _positional_embedding(L, EMBED)
    info = plsc.get_sparse_core_info()
    n_workers = info.num_cores * info.num_subcores
    out = _make_sc_kernel(n_workers)(seq, token_table, pe)
    return out.reshape(B, L, EMBED)


# speedup vs baseline: 2.6422x; 1.1389x over previous
import functools
import math

import jax
import jax.numpy as jnp
from jax import lax
from jax.experimental import pallas as pl
from jax.experimental.pallas import tpu as pltpu
from jax.experimental.pallas import tpu_sc as plsc

VOCAB = 100000
EMBED = 128
B = 1024
L = 200
LANES = 16
CHUNKS_PER_ROW = EMBED // LANES
GCHUNK = 40
PE_PERIOD = L // GCHUNK
NBUF = 6
LOOKAHEAD = 4


def _positional_embedding(seq_len, d_model):
    position = jnp.arange(0, seq_len, dtype=jnp.float32)[:, None]
    div_term = jnp.exp(
        jnp.arange(0, d_model, 2, dtype=jnp.float32) * -(math.log(10000.0) / d_model)
    )
    pe = jnp.zeros((seq_len, d_model), dtype=jnp.float32)
    pe = pe.at[:, 0::2].set(jnp.sin(position * div_term))
    pe = pe.at[:, 1::2].set(jnp.cos(position * div_term))
    return pe


def _make_sc_kernel(n_workers):
    n_chunks = (B * L) // GCHUNK
    chunks_per_w = n_chunks // n_workers
    mesh = plsc.VectorSubcoreMesh(core_axis_name="c", subcore_axis_name="s")
    num_cores = mesh.num_cores

    @functools.partial(
        pl.kernel,
        mesh=mesh,
        out_type=jax.ShapeDtypeStruct((B * L, EMBED), jnp.float32),
        scratch_types=(
            [pltpu.VMEM((chunks_per_w, GCHUNK), jnp.int32)]
            + [pltpu.VMEM((L, EMBED), jnp.float32)]
            + [pltpu.VMEM((GCHUNK, EMBED), jnp.float32)] * NBUF
            + [pltpu.SemaphoreType.DMA] * (2 * NBUF)
        ),
    )
    def k(seq_hbm, table_hbm, pe_hbm, out_hbm, idx_v, pe_v, *bufs_sems):
        rows = bufs_sems[:NBUF]
        gsem = bufs_sems[NBUF : 2 * NBUF]
        osem = bufs_sems[2 * NBUF :]
        wid = lax.axis_index("s") * num_cores + lax.axis_index("c")
        chunk_base = wid * chunks_per_w
        pltpu.sync_copy(pe_hbm, pe_v)
        pltpu.sync_copy(seq_hbm.at[pl.ds(chunk_base, chunks_per_w)], idx_v)

        def gather_start(c, b):
            pltpu.make_async_copy(
                table_hbm.at[idx_v.at[c]], rows[b], gsem[b]
            ).start()

        def gather_wait(c, b):
            pltpu.make_async_copy(
                table_hbm.at[idx_v.at[c]], rows[b], gsem[b]
            ).wait()

        def add_pe(c, b):
            pe_off = lax.rem(c, PE_PERIOD) * GCHUNK

            @plsc.parallel_loop(0, GCHUNK, unroll=1)
            def add_row(r):
                for j in range(CHUNKS_PER_ROW):
                    sl = pl.ds(j * LANES, LANES)
                    plsc.addupdate(rows[b].at[r, sl], pe_v[pe_off + r, sl])

        def out_start(c, b):
            pltpu.make_async_copy(
                rows[b], out_hbm.at[pl.ds((chunk_base + c) * GCHUNK, GCHUNK)], osem[b]
            ).start()

        def out_wait(b):
            pltpu.make_async_copy(
                rows[b], out_hbm.at[pl.ds(0, GCHUNK)], osem[b]
            ).wait()

        def slot(c, pb, bslot, wait_out, issue):
            if issue:
                if wait_out:
                    out_wait(bslot)
                gather_start(c + LOOKAHEAD, bslot)
            gather_wait(c, pb)
            add_pe(c, pb)
            out_start(c, pb)

        for b in range(LOOKAHEAD):
            gather_start(b, b)

        c0 = NBUF - LOOKAHEAD
        for c in range(c0):
            slot(c, c % NBUF, (c + LOOKAHEAD) % NBUF, False, True)

        n_main = (chunks_per_w - LOOKAHEAD - c0) // NBUF

        def group(g0, carry):
            for b in range(NBUF):
                c = c0 + g0 * NBUF + b
                slot(c, (c0 + b) % NBUF, (c0 + b + LOOKAHEAD) % NBUF, True, True)
            return carry

        lax.fori_loop(0, n_main, group, 0)

        for c in range(c0 + n_main * NBUF, chunks_per_w):
            slot(c, c % NBUF, (c + LOOKAHEAD) % NBUF,
                 c + LOOKAHEAD >= NBUF, c + LOOKAHEAD < chunks_per_w)

        for b in range(NBUF):
            out_wait(b)

    return k


def kernel(sequence, token_table):
    n_chunks = (B * L) // GCHUNK
    seq = sequence.astype(jnp.int32).reshape(n_chunks, GCHUNK)
    pe = _positional_embedding(L, EMBED)
    info = plsc.get_sparse_core_info()
    n_workers = info.num_cores * info.num_subcores
    out = _make_sc_kernel(n_workers)(seq, token_table, pe)
    return out.reshape(B, L, EMBED)


# speedup vs baseline: 2.6624x; 1.0076x over previous
import functools
import math

import jax
import jax.numpy as jnp
from jax import lax
from jax.experimental import pallas as pl
from jax.experimental.pallas import tpu as pltpu
from jax.experimental.pallas import tpu_sc as plsc

VOCAB = 100000
EMBED = 128
B = 1024
L = 200
LANES = 16
CHUNKS_PER_ROW = EMBED // LANES
GCHUNK = 40
PE_PERIOD = L // GCHUNK
NBUF = 8
LOOKAHEAD = 6


def _positional_embedding(seq_len, d_model):
    position = jnp.arange(0, seq_len, dtype=jnp.float32)[:, None]
    div_term = jnp.exp(
        jnp.arange(0, d_model, 2, dtype=jnp.float32) * -(math.log(10000.0) / d_model)
    )
    pe = jnp.zeros((seq_len, d_model), dtype=jnp.float32)
    pe = pe.at[:, 0::2].set(jnp.sin(position * div_term))
    pe = pe.at[:, 1::2].set(jnp.cos(position * div_term))
    return pe


def _make_sc_kernel(n_workers):
    n_chunks = (B * L) // GCHUNK
    chunks_per_w = n_chunks // n_workers
    mesh = plsc.VectorSubcoreMesh(core_axis_name="c", subcore_axis_name="s")
    num_cores = mesh.num_cores

    @functools.partial(
        pl.kernel,
        mesh=mesh,
        out_type=jax.ShapeDtypeStruct((B * L, EMBED), jnp.float32),
        scratch_types=(
            [pltpu.VMEM((chunks_per_w, GCHUNK), jnp.int32)]
            + [pltpu.VMEM((L, EMBED), jnp.float32)]
            + [pltpu.VMEM((GCHUNK, EMBED), jnp.float32)] * NBUF
            + [pltpu.SemaphoreType.DMA] * (2 * NBUF)
        ),
    )
    def k(seq_hbm, table_hbm, pe_hbm, out_hbm, idx_v, pe_v, *bufs_sems):
        rows = bufs_sems[:NBUF]
        gsem = bufs_sems[NBUF : 2 * NBUF]
        osem = bufs_sems[2 * NBUF :]
        wid = lax.axis_index("s") * num_cores + lax.axis_index("c")
        chunk_base = wid * chunks_per_w
        pltpu.sync_copy(pe_hbm, pe_v)
        pltpu.sync_copy(seq_hbm.at[pl.ds(chunk_base, chunks_per_w)], idx_v)

        def gather_start(c, b):
            pltpu.make_async_copy(
                table_hbm.at[idx_v.at[c]], rows[b], gsem[b]
            ).start()

        def gather_wait(c, b):
            pltpu.make_async_copy(
                table_hbm.at[idx_v.at[c]], rows[b], gsem[b]
            ).wait()

        def add_pe(c, b):
            pe_off = lax.rem(c, PE_PERIOD) * GCHUNK

            @plsc.parallel_loop(0, GCHUNK, unroll=1)
            def add_row(r):
                for j in range(CHUNKS_PER_ROW):
                    sl = pl.ds(j * LANES, LANES)
                    plsc.addupdate(rows[b].at[r, sl], pe_v[pe_off + r, sl])

        def out_start(c, b):
            pltpu.make_async_copy(
                rows[b], out_hbm.at[pl.ds((chunk_base + c) * GCHUNK, GCHUNK)], osem[b]
            ).start()

        def out_wait(b):
            pltpu.make_async_copy(
                rows[b], out_hbm.at[pl.ds(0, GCHUNK)], osem[b]
            ).wait()

        def slot(c, pb, bslot, wait_out, issue):
            if issue:
                if wait_out:
                    out_wait(bslot)
                gather_start(c + LOOKAHEAD, bslot)
            gather_wait(c, pb)
            add_pe(c, pb)
            out_start(c, pb)

        for b in range(LOOKAHEAD):
            gather_start(b, b)

        c0 = NBUF - LOOKAHEAD
        for c in range(c0):
            slot(c, c % NBUF, (c + LOOKAHEAD) % NBUF, False, True)

        n_main = (chunks_per_w - LOOKAHEAD - c0) // NBUF

        def group(g0, carry):
            for b in range(NBUF):
                c = c0 + g0 * NBUF + b
                slot(c, (c0 + b) % NBUF, (c0 + b + LOOKAHEAD) % NBUF, True, True)
            return carry

        lax.fori_loop(0, n_main, group, 0)

        for c in range(c0 + n_main * NBUF, chunks_per_w):
            slot(c, c % NBUF, (c + LOOKAHEAD) % NBUF,
                 c + LOOKAHEAD >= NBUF, c + LOOKAHEAD < chunks_per_w)

        for b in range(NBUF):
            out_wait(b)

    return k


def kernel(sequence, token_table):
    n_chunks = (B * L) // GCHUNK
    seq = sequence.astype(jnp.int32).reshape(n_chunks, GCHUNK)
    pe = _positional_embedding(L, EMBED)
    info = plsc.get_sparse_core_info()
    n_workers = info.num_cores * info.num_subcores
    out = _make_sc_kernel(n_workers)(seq, token_table, pe)
    return out.reshape(B, L, EMBED)


# speedup vs baseline: 2.6670x; 1.0017x over previous
import functools
import math

import jax
import jax.numpy as jnp
from jax import lax
from jax.experimental import pallas as pl
from jax.experimental.pallas import tpu as pltpu
from jax.experimental.pallas import tpu_sc as plsc

VOCAB = 100000
EMBED = 128
B = 1024
L = 200
LANES = 16
CHUNKS_PER_ROW = EMBED // LANES
GCHUNK = 128
NBUF = 5
LOOKAHEAD = 3


def _positional_embedding(seq_len, d_model):
    position = jnp.arange(0, seq_len, dtype=jnp.float32)[:, None]
    div_term = jnp.exp(
        jnp.arange(0, d_model, 2, dtype=jnp.float32) * -(math.log(10000.0) / d_model)
    )
    pe = jnp.zeros((seq_len, d_model), dtype=jnp.float32)
    pe = pe.at[:, 0::2].set(jnp.sin(position * div_term))
    pe = pe.at[:, 1::2].set(jnp.cos(position * div_term))
    return pe


def _make_sc_kernel(n_workers):
    n_chunks = (B * L) // GCHUNK
    chunks_per_w = n_chunks // n_workers
    mesh = plsc.VectorSubcoreMesh(core_axis_name="c", subcore_axis_name="s")
    num_cores = mesh.num_cores

    @functools.partial(
        pl.kernel,
        mesh=mesh,
        out_type=jax.ShapeDtypeStruct((B * L, EMBED), jnp.float32),
        scratch_types=(
            [pltpu.VMEM((chunks_per_w, GCHUNK), jnp.int32)]
            + [pltpu.VMEM((L, EMBED), jnp.float32)]
            + [pltpu.VMEM((GCHUNK, EMBED), jnp.float32)] * NBUF
            + [pltpu.SemaphoreType.DMA] * (2 * NBUF)
        ),
    )
    def k(seq_hbm, table_hbm, pe_hbm, out_hbm, idx_v, pe_v, *bufs_sems):
        rows = bufs_sems[:NBUF]
        gsem = bufs_sems[NBUF : 2 * NBUF]
        osem = bufs_sems[2 * NBUF :]
        wid = lax.axis_index("s") * num_cores + lax.axis_index("c")
        chunk_base = wid * chunks_per_w
        pltpu.sync_copy(pe_hbm, pe_v)
        pltpu.sync_copy(seq_hbm.at[wid], idx_v)

        def gather_start(c, b):
            pltpu.make_async_copy(
                table_hbm.at[idx_v.at[c]], rows[b], gsem[b]
            ).start()

        def gather_wait(c, b):
            pltpu.make_async_copy(
                table_hbm.at[idx_v.at[c]], rows[b], gsem[b]
            ).wait()

        def add_pe(c, b):
            pe_off = lax.rem((chunk_base + c) * GCHUNK, L)

            @plsc.parallel_loop(0, GCHUNK, unroll=1)
            def add_row(r):
                pr = pe_off + r
                pr = jnp.where(pr >= L, pr - L, pr)
                for j in range(CHUNKS_PER_ROW):
                    sl = pl.ds(j * LANES, LANES)
                    plsc.addupdate(rows[b].at[r, sl], pe_v[pr, sl])

        def out_start(c, b):
            pltpu.make_async_copy(
                rows[b], out_hbm.at[pl.ds((chunk_base + c) * GCHUNK, GCHUNK)], osem[b]
            ).start()

        def out_wait(b):
            pltpu.make_async_copy(
                rows[b], out_hbm.at[pl.ds(0, GCHUNK)], osem[b]
            ).wait()

        def slot(c, pb, bslot, wait_out, issue):
            if issue:
                if wait_out:
                    out_wait(bslot)
                gather_start(c + LOOKAHEAD, bslot)
            gather_wait(c, pb)
            add_pe(c, pb)
            out_start(c, pb)

        for b in range(LOOKAHEAD):
            gather_start(b, b)

        c0 = NBUF - LOOKAHEAD
        for c in range(c0):
            slot(c, c % NBUF, (c + LOOKAHEAD) % NBUF, False, True)

        n_main = (chunks_per_w - LOOKAHEAD - c0) // NBUF

        def group(g0, carry):
            for b in range(NBUF):
                c = c0 + g0 * NBUF + b
                slot(c, (c0 + b) % NBUF, (c0 + b + LOOKAHEAD) % NBUF, True, True)
            return carry

        lax.fori_loop(0, n_main, group, 0)

        for c in range(c0 + n_main * NBUF, chunks_per_w):
            slot(c, c % NBUF, (c + LOOKAHEAD) % NBUF,
                 c + LOOKAHEAD >= NBUF, c + LOOKAHEAD < chunks_per_w)

        for b in range(NBUF):
            out_wait(b)

    return k


def kernel(sequence, token_table):
    info = plsc.get_sparse_core_info()
    n_workers = info.num_cores * info.num_subcores
    n_chunks = (B * L) // GCHUNK
    seq = sequence.astype(jnp.int32).reshape(n_workers, n_chunks // n_workers, GCHUNK)
    pe = _positional_embedding(L, EMBED)
    out = _make_sc_kernel(n_workers)(seq, token_table, pe)
    return out.reshape(B, L, EMBED)


# speedup vs baseline: 2.6670x; 1.0000x over previous
import functools
import math

import jax
import jax.numpy as jnp
from jax import lax
from jax.experimental import pallas as pl
from jax.experimental.pallas import tpu as pltpu
from jax.experimental.pallas import tpu_sc as plsc

VOCAB = 100000
EMBED = 128
B = 1024
L = 200
LANES = 16
CHUNKS_PER_ROW = EMBED // LANES
GCHUNK = 128
NBUF = 5
LOOKAHEAD = 3


def _positional_embedding(seq_len, d_model):
    position = jnp.arange(0, seq_len, dtype=jnp.float32)[:, None]
    div_term = jnp.exp(
        jnp.arange(0, d_model, 2, dtype=jnp.float32) * -(math.log(10000.0) / d_model)
    )
    pe = jnp.zeros((seq_len, d_model), dtype=jnp.float32)
    pe = pe.at[:, 0::2].set(jnp.sin(position * div_term))
    pe = pe.at[:, 1::2].set(jnp.cos(position * div_term))
    return pe


def _make_sc_kernel(n_workers):
    n_chunks = (B * L) // GCHUNK
    chunks_per_w = n_chunks // n_workers
    mesh = plsc.VectorSubcoreMesh(core_axis_name="c", subcore_axis_name="s")
    num_cores = mesh.num_cores

    @functools.partial(
        pl.kernel,
        mesh=mesh,
        compiler_params=pltpu.CompilerParams(use_tc_tiling_on_sc=False),
        out_type=jax.ShapeDtypeStruct((B * L, EMBED), jnp.float32),
        scratch_types=(
            [pltpu.VMEM((chunks_per_w, GCHUNK), jnp.int32)]
            + [pltpu.VMEM((L, EMBED), jnp.float32)]
            + [pltpu.VMEM((GCHUNK, EMBED), jnp.float32)] * NBUF
            + [pltpu.SemaphoreType.DMA] * (2 * NBUF)
        ),
    )
    def k(seq_hbm, table_hbm, pe_hbm, out_hbm, idx_v, pe_v, *bufs_sems):
        rows = bufs_sems[:NBUF]
        gsem = bufs_sems[NBUF : 2 * NBUF]
        osem = bufs_sems[2 * NBUF :]
        wid = lax.axis_index("s") * num_cores + lax.axis_index("c")
        chunk_base = wid * chunks_per_w
        pltpu.sync_copy(pe_hbm, pe_v)
        pltpu.sync_copy(seq_hbm.at[wid], idx_v)

        def gather_start(c, b):
            pltpu.make_async_copy(
                table_hbm.at[idx_v.at[c]], rows[b], gsem[b]
            ).start()

        def gather_wait(c, b):
            pltpu.make_async_copy(
                table_hbm.at[idx_v.at[c]], rows[b], gsem[b]
            ).wait()

        def add_pe(c, b):
            pe_off = lax.rem((chunk_base + c) * GCHUNK, L)

            @plsc.parallel_loop(0, GCHUNK, unroll=1)
            def add_row(r):
                pr = pe_off + r
                pr = jnp.where(pr >= L, pr - L, pr)
                for j in range(CHUNKS_PER_ROW):
                    sl = pl.ds(j * LANES, LANES)
                    plsc.addupdate(rows[b].at[r, sl], pe_v[pr, sl])

        def out_start(c, b):
            pltpu.make_async_copy(
                rows[b], out_hbm.at[pl.ds((chunk_base + c) * GCHUNK, GCHUNK)], osem[b]
            ).start()

        def out_wait(b):
            pltpu.make_async_copy(
                rows[b], out_hbm.at[pl.ds(0, GCHUNK)], osem[b]
            ).wait()

        def slot(c, pb, bslot, wait_out, issue):
            if issue:
                if wait_out:
                    out_wait(bslot)
                gather_start(c + LOOKAHEAD, bslot)
            gather_wait(c, pb)
            add_pe(c, pb)
            out_start(c, pb)

        for b in range(LOOKAHEAD):
            gather_start(b, b)

        c0 = NBUF - LOOKAHEAD
        for c in range(c0):
            slot(c, c % NBUF, (c + LOOKAHEAD) % NBUF, False, True)

        n_main = (chunks_per_w - LOOKAHEAD - c0) // NBUF

        def group(g0, carry):
            for b in range(NBUF):
                c = c0 + g0 * NBUF + b
                slot(c, (c0 + b) % NBUF, (c0 + b + LOOKAHEAD) % NBUF, True, True)
            return carry

        lax.fori_loop(0, n_main, group, 0)

        for c in range(c0 + n_main * NBUF, chunks_per_w):
            slot(c, c % NBUF, (c + LOOKAHEAD) % NBUF,
                 c + LOOKAHEAD >= NBUF, c + LOOKAHEAD < chunks_per_w)

        for b in range(NBUF):
            out_wait(b)

    return k


def kernel(sequence, token_table):
    info = plsc.get_sparse_core_info()
    n_workers = info.num_cores * info.num_subcores
    n_chunks = (B * L) // GCHUNK
    seq = sequence.astype(jnp.int32).reshape(n_workers, n_chunks // n_workers, GCHUNK)
    pe = _positional_embedding(L, EMBED)
    out = _make_sc_kernel(n_workers)(seq, token_table, pe)
    return out.reshape(B, L, EMBED)
